# baseline (speedup 1.0000x reference)
import functools

import jax
import jax.numpy as jnp
from jax import lax
from jax.experimental import pallas as pl
from jax.experimental.pallas import tpu as pltpu
from jax.experimental.pallas import tpu_sc as plsc

N = 10000
E = 160000
NFEAT = 128
NHID = 128
NCLASS = 64

NC = 2
NS = 16
NW = NC * NS
CHUNK = 128
CPT = 40
TOTCH = 1280
EPAD = NW * CPT * CHUNK
NP = 10112
RPT = NP // NS

_MESH = plsc.VectorSubcoreMesh(core_axis_name="c", subcore_axis_name="s")

MBLK = 2000


DEGW = 128


def _sc_degree(dst_tiles, ones_rows, zeros_deg):
    @functools.partial(
        pl.kernel,
        out_type=jax.ShapeDtypeStruct((NC, NP, DEGW), jnp.float32),
        mesh=_MESH,
        scratch_types=[
            pltpu.VMEM((CPT, CHUNK), jnp.int32),
            pltpu.VMEM((CHUNK, DEGW), jnp.float32),
            pltpu.VMEM_SHARED((NP, DEGW), jnp.float32),
            pltpu.SemaphoreType.DMA,
            pltpu.SemaphoreType.DMA,
        ],
    )
    def deg_kernel(dst_hbm, ones_hbm, zeros_hbm, out_hbm, idx_v, ones_v, acc_s,
                   s0, s1):
        cid = lax.axis_index("c")
        sid = lax.axis_index("s")
        gtile = cid * NS + sid
        base = sid * RPT
        pltpu.sync_copy(zeros_hbm.at[pl.ds(base, RPT)], acc_s.at[pl.ds(base, RPT)])
        pltpu.sync_copy(ones_hbm, ones_v)
        pltpu.sync_copy(dst_hbm.at[pl.ds(gtile * CPT, CPT)], idx_v)
        plsc.subcore_barrier()

        @pl.loop(0, CPT)
        def _(j):
            pltpu.async_copy(ones_v, acc_s.at[idx_v.at[j]], s0, add=True)

        @pl.loop(0, CPT)
        def _(j):
            pltpu.make_async_copy(ones_v, acc_s.at[idx_v.at[j]], s0).wait()

        plsc.subcore_barrier()
        pltpu.sync_copy(acc_s.at[pl.ds(base, RPT)],
                        out_hbm.at[cid, pl.ds(base, RPT)])

    return deg_kernel(dst_tiles, ones_rows, zeros_deg)


def _sc_edge_pass(hs, src_flat, dst3, zeros128):
    NBUF = 2
    C0T = TOTCH // NS
    NGRP = C0T // NBUF

    @functools.partial(
        pl.kernel,
        out_type=jax.ShapeDtypeStruct((NP, NHID), jnp.float32),
        mesh=_MESH,
        scratch_types=(
            [pltpu.VMEM((C0T, CHUNK), jnp.int32)]
            + [pltpu.VMEM((1, CHUNK), jnp.int32)] * NBUF
            + [pltpu.VMEM((CHUNK, NHID), jnp.float32)] * NBUF
            + [pltpu.SemaphoreType.DMA] * (3 * NBUF)
            + [pltpu.VMEM_SHARED((NP, NHID), jnp.float32)]
        ),
    )
    def edge_kernel(hs_hbm, src_hbm, dst_hbm, zeros_hbm, out_hbm,
                    src_v, *rest):
        dstib = rest[:NBUF]
        bufs = rest[NBUF:2 * NBUF]
        gsem = rest[2 * NBUF:3 * NBUF]
        ssem = rest[3 * NBUF:4 * NBUF]
        dsem = rest[4 * NBUF:5 * NBUF]
        acc_s = rest[5 * NBUF]
        cid = lax.axis_index("c")
        sid = lax.axis_index("s")

        @pl.when(cid == 0)
        def _():
            base = sid * RPT
            chunk0 = sid * C0T
            pltpu.sync_copy(zeros_hbm.at[pl.ds(base, RPT)],
                            acc_s.at[pl.ds(base, RPT)])
            pltpu.sync_copy(src_hbm.at[pl.ds(chunk0, C0T)], src_v)
            plsc.subcore_barrier()

            def fire_gather(j, k):
                pltpu.async_copy(hs_hbm.at[src_v.at[j]], bufs[k], gsem[k])

            def wait_gather(j, k):
                pltpu.make_async_copy(hs_hbm.at[src_v.at[j]], bufs[k],
                                      gsem[k]).wait()

            def fire_dst(j, k):
                pltpu.async_copy(dst_hbm.at[chunk0 + j], dstib[k], dsem[k])

            def wait_dst(j, k):
                pltpu.make_async_copy(dst_hbm.at[chunk0 + j], dstib[k],
                                      dsem[k]).wait()

            def fire_scatter(k):
                pltpu.async_copy(bufs[k], acc_s.at[dstib[k].at[0]], ssem[k],
                                 add=True)

            def wait_scatter(k):
                pltpu.make_async_copy(bufs[k], acc_s.at[dstib[k].at[0]],
                                      ssem[k]).wait()

            for k in range(NBUF):
                fire_dst(k, k)
                fire_gather(k, k)

            @pl.loop(1, NGRP)
            def _(i):
                j0 = i * NBUF
                for k in range(NBUF):
                    wait_gather(j0 - NBUF + k, k)
                    wait_dst(j0 - NBUF + k, k)
                    fire_scatter(k)
                for k in range(NBUF):
                    wait_scatter(k)
                    fire_dst(j0 + k, k)
                    fire_gather(j0 + k, k)

            for k in range(NBUF):
                wait_gather(C0T - NBUF + k, k)
                wait_dst(C0T - NBUF + k, k)
                fire_scatter(k)
            for k in range(NBUF):
                wait_scatter(k)

            plsc.subcore_barrier()
            pltpu.sync_copy(acc_s.at[pl.ds(base, RPT)],
                            out_hbm.at[pl.ds(base, RPT)])

    return edge_kernel(hs, src_flat, dst3, zeros128)


def _dot(a, b):
    return jnp.dot(a, b, preferred_element_type=jnp.float32,
                   precision=lax.Precision.HIGHEST)


def _row_spec(cols):
    return pl.BlockSpec((MBLK, cols), lambda i: (i, 0))


def _full_spec(rows, cols):
    return pl.BlockSpec((rows, cols), lambda i: (0, 0))


def _tc_encode(x, enc_W, enc_b):
    def body(x_ref, w_ref, b_ref, o_ref):
        o_ref[...] = jnp.maximum(_dot(x_ref[...], w_ref[...]) + b_ref[...], 0.0)

    return pl.pallas_call(
        body,
        grid=(N // MBLK,),
        in_specs=[_row_spec(NFEAT), _full_spec(NFEAT, NHID), _full_spec(1, NHID)],
        out_specs=_row_spec(NHID),
        out_shape=jax.ShapeDtypeStruct((N, NHID), jnp.float32),
    )(x, enc_W, enc_b.reshape(1, NHID))


def _tc_scale_matmul(x0, conv_W, dega, degb):
    def body(x_ref, w_ref, da_ref, db_ref, o_ref):
        dinv = lax.rsqrt(da_ref[:, 0:1] + db_ref[:, 0:1] + 1.0)
        o_ref[...] = dinv * _dot(x_ref[...], w_ref[...])

    return pl.pallas_call(
        body,
        grid=(N // MBLK,),
        in_specs=[_row_spec(NHID), _full_spec(NHID, NHID),
                  _row_spec(16), _row_spec(16)],
        out_specs=_row_spec(NHID),
        out_shape=jax.ShapeDtypeStruct((N, NHID), jnp.float32),
    )(x0, conv_W, dega, degb)


def _tc_post_and_next(acc, hs, x_prev, conv_b, conv_W, dega, degb):
    def body(a_ref, hs_ref, xp_ref, b_ref, w_ref, da_ref, db_ref,
             x_ref, hsn_ref):
        dinv = lax.rsqrt(da_ref[:, 0:1] + db_ref[:, 0:1] + 1.0)
        agg = a_ref[...] + hs_ref[...]
        x_new = jnp.maximum(dinv * agg + b_ref[...] + xp_ref[...], 0.0)
        x_ref[...] = x_new
        hsn_ref[...] = dinv * _dot(x_new, w_ref[...])

    return pl.pallas_call(
        body,
        grid=(N // MBLK,),
        in_specs=[_row_spec(NHID), _row_spec(NHID),
                  _row_spec(NHID), _full_spec(1, NHID), _full_spec(NHID, NHID),
                  _row_spec(16), _row_spec(16)],
        out_specs=[_row_spec(NHID), _row_spec(NHID)],
        out_shape=[jax.ShapeDtypeStruct((N, NHID), jnp.float32),
                   jax.ShapeDtypeStruct((N, NHID), jnp.float32)],
    )(acc, hs, x_prev, conv_b.reshape(1, NHID), conv_W, dega, degb)


def _tc_post_and_decode(acc, hs, x_prev, conv_b, dec_W, dec_b, dega, degb):
    def body(a_ref, hs_ref, xp_ref, b_ref, w_ref, db2_ref, da_ref,
             db_ref, o_ref):
        dinv = lax.rsqrt(da_ref[:, 0:1] + db_ref[:, 0:1] + 1.0)
        agg = a_ref[...] + hs_ref[...]
        x_new = jnp.maximum(dinv * agg + b_ref[...] + xp_ref[...], 0.0)
        o_ref[...] = _dot(x_new, w_ref[...]) + db2_ref[...]

    return pl.pallas_call(
        body,
        grid=(N // MBLK,),
        in_specs=[_row_spec(NHID), _row_spec(NHID),
                  _row_spec(NHID), _full_spec(1, NHID), _full_spec(NHID, NCLASS),
                  _full_spec(1, NCLASS), _row_spec(16), _row_spec(16)],
        out_specs=_row_spec(NCLASS),
        out_shape=jax.ShapeDtypeStruct((N, NCLASS), jnp.float32),
    )(acc, hs, x_prev, conv_b.reshape(1, NHID), dec_W,
      dec_b.reshape(1, NCLASS), dega, degb)


def kernel(x, edge_index, enc_W, enc_b, conv_W, conv_b, dec_W, dec_b):
    src = edge_index[0]
    dst = edge_index[1]
    pad = EPAD - E
    srcp = jnp.concatenate([src, jnp.zeros((pad,), jnp.int32)])
    dstp = jnp.concatenate([dst, jnp.full((pad,), N, jnp.int32)])
    src_flat = srcp.reshape(TOTCH, CHUNK)
    dst_flat = dstp.reshape(TOTCH, CHUNK)
    dst3 = dstp.reshape(TOTCH, 1, CHUNK)

    zeros128 = jnp.zeros((NP, NHID), jnp.float32)
    ones_rows = jnp.ones((CHUNK, DEGW), jnp.float32)

    degs = _sc_degree(dst_flat, ones_rows, zeros128)
    dega = degs[0, :N, :16]
    degb = degs[1, :N, :16]

    x0 = _tc_encode(x, enc_W, enc_b)
    hs1 = _tc_scale_matmul(x0, conv_W, dega, degb)

    acc = _sc_edge_pass(hs1, src_flat, dst3, zeros128)
    x1, hs2 = _tc_post_and_next(acc[:N], hs1, x0,
                                conv_b, conv_W, dega, degb)

    acc2 = _sc_edge_pass(hs2, src_flat, dst3, zeros128)
    out = _tc_post_and_decode(acc2[:N], hs2, x1,
                              conv_b, dec_W, dec_b, dega, degb)
    return out
